# baseline (speedup 1.0000x reference)
import functools

import jax
import jax.numpy as jnp
from jax import lax
from jax.experimental import pallas as pl
from jax.experimental.pallas import tpu as pltpu
from jax.experimental.pallas import tpu_sc as plsc

N = 10000
E = 160000
B = 128
NB = 80
NBD = 40
EP = 163840
NACC = 10240
STRIPE = NACC // 16
F = 128
R = 1000
G = N // R

_mesh = plsc.VectorSubcoreMesh(
    core_axis_name="c", subcore_axis_name="s", num_cores=2, num_subcores=16)


BG = 64
NBUF = 4
NBG = EP // (16 * BG)
QN = 32
NQ = NBG // QN


def _make_prop(nchunk):
    chunks_of_core = {
        0: tuple(c for c in range(nchunk) if c % 2 == 0),
        1: tuple(c for c in range(nchunk) if c % 2 == 1),
    }

    @functools.partial(
        pl.kernel,
        out_type=jax.ShapeDtypeStruct((nchunk, NACC, F), jnp.float32),
        mesh=_mesh,
        scratch_types=[
            pltpu.VMEM_SHARED((NACC, F), jnp.float32),
            pltpu.VMEM((QN, BG), jnp.int32),
            pltpu.VMEM((QN, BG), jnp.int32),
            [pltpu.VMEM((BG, F), jnp.float32) for _ in range(NBUF)],
            [pltpu.SemaphoreType.DMA for _ in range(NBUF)],
            [pltpu.SemaphoreType.DMA for _ in range(NBUF)],
        ],
    )
    def prop(xws_hbm, rowc_hbm, col_hbm, zeros_hbm, out_hbm,
             acc_sh, row_v, col_v, bufs, gsems, ssems):
        cid = lax.axis_index("c")
        sid = lax.axis_index("s")
        base = sid * STRIPE
        for core in (0, 1):
            for c in chunks_of_core[core]:
                @pl.when(cid == core)
                def _(c=c):
                    pltpu.sync_copy(zeros_hbm, bufs[0])
                    for z in range(STRIPE // BG):
                        pltpu.sync_copy(
                            bufs[0], acc_sh.at[pl.ds(base + z * BG, BG)])
                    plsc.subcore_barrier()
                    for q in range(NQ):
                        pltpu.sync_copy(
                            rowc_hbm.at[c, sid, pl.ds(q * QN, QN)], row_v)
                        pltpu.sync_copy(
                            col_hbm.at[sid, pl.ds(q * QN, QN)], col_v)

                        def group(i, carry):
                            j = NBUF * i
                            gds = [
                                pltpu.async_copy(
                                    xws_hbm.at[row_v.at[j + k]],
                                    bufs[k], gsems[k])
                                for k in range(NBUF)]
                            sds = []
                            for k in range(NBUF):
                                gds[k].wait()
                                sds.append(pltpu.async_copy(
                                    bufs[k], acc_sh.at[col_v.at[j + k]],
                                    ssems[k], add=True))
                            for k in range(NBUF):
                                sds[k].wait()
                            return carry

                        lax.fori_loop(0, QN // NBUF, group, 0)
                    plsc.subcore_barrier()
                    for z in range(STRIPE // BG):
                        pltpu.sync_copy(
                            acc_sh.at[pl.ds(base + z * BG, BG)], bufs[0])
                        pltpu.sync_copy(
                            bufs[0], out_hbm.at[c, pl.ds(base + z * BG, BG)])

    return prop


_prop4 = _make_prop(4)
_prop2 = _make_prop(2)


@functools.partial(
    pl.kernel,
    out_type=jax.ShapeDtypeStruct((2, NACC, F), jnp.float32),
    mesh=_mesh,
    scratch_types=[
        pltpu.VMEM_SHARED((NACC, F), jnp.float32),
        pltpu.VMEM((NBD, B), jnp.int32),
        pltpu.VMEM((B, F), jnp.float32),
        pltpu.VMEM((B, F), jnp.float32),
    ],
)
def _deg(col32_hbm, ones_hbm, zeros_hbm, out_hbm, acc_sh, col_v, ones_v, wb_v):
    cid = lax.axis_index("c")
    sid = lax.axis_index("s")
    wid = sid * 2 + cid
    base = sid * STRIPE
    pltpu.sync_copy(col32_hbm.at[wid], col_v)
    pltpu.sync_copy(zeros_hbm, wb_v)
    for z in range(STRIPE // B):
        pltpu.sync_copy(wb_v, acc_sh.at[pl.ds(base + z * B, B)])
    pltpu.sync_copy(ones_hbm, ones_v)
    plsc.subcore_barrier()

    def body(j, carry):
        pltpu.sync_copy(ones_v, acc_sh.at[col_v.at[j]], add=True)
        return carry

    lax.fori_loop(0, NBD, body, 0)
    plsc.subcore_barrier()
    for z in range(STRIPE // B):
        pltpu.sync_copy(acc_sh.at[pl.ds(base + z * B, B)], wb_v)
        pltpu.sync_copy(wb_v, out_hbm.at[cid, pl.ds(base + z * B, B)])


def _mlp_body(x_ref, w0_ref, b0_ref, a0_ref, w1_ref, b1_ref, wg0_ref, deg_ref,
              out_ref, dinv_ref):
    x = x_ref[...]
    a = a0_ref[0, 0]
    h = jnp.dot(x, w0_ref[...], preferred_element_type=jnp.float32) + b0_ref[...]
    h = jnp.where(h >= 0, h, a * h)
    h = jnp.dot(h, w1_ref[...], preferred_element_type=jnp.float32) + b1_ref[...]
    y = jnp.dot(h, wg0_ref[...], preferred_element_type=jnp.float32)
    dinv = lax.rsqrt(deg_ref[0, :, 0:1] + deg_ref[1, :, 0:1] + 1.0)
    dinv_ref[...] = dinv
    y = y * dinv
    for c in range(4):
        out_ref[c] = y[:, c * F:(c + 1) * F]


def _tc_mlp(X, W0, b0, a0, W1, b1, Wg0, degp):
    return pl.pallas_call(
        _mlp_body,
        grid=(G,),
        in_specs=[
            pl.BlockSpec((R, 256), lambda i: (i, 0)),
            pl.BlockSpec((256, 512), lambda i: (0, 0)),
            pl.BlockSpec((512,), lambda i: (0,)),
            pl.BlockSpec((1, 1), lambda i: (0, 0)),
            pl.BlockSpec((512, 512), lambda i: (0, 0)),
            pl.BlockSpec((512,), lambda i: (0,)),
            pl.BlockSpec((512, 512), lambda i: (0, 0)),
            pl.BlockSpec((2, R, F), lambda i: (0, i, 0)),
        ],
        out_specs=[
            pl.BlockSpec((4, R, F), lambda i: (0, i, 0)),
            pl.BlockSpec((R, 1), lambda i: (i, 0)),
        ],
        out_shape=[
            jax.ShapeDtypeStruct((4, N, F), jnp.float32),
            jax.ShapeDtypeStruct((N, 1), jnp.float32),
        ],
        compiler_params=pltpu.CompilerParams(
            dimension_semantics=("arbitrary",)),
    )(X, W0, b0, a0, W1, b1, Wg0, degp)


def _make_mid(nc_out):
    def mid_body(s_ref, xw_ref, dinv_ref, b_ref, a_ref, w_ref, out_ref):
        dinv = dinv_ref[...]
        a = a_ref[0, 0]
        parts = []
        for c in range(4):
            t = (s_ref[c] + xw_ref[c]) * dinv + b_ref[pl.ds(c * F, F)]
            parts.append(jnp.where(t >= 0, t, a * t))
        h = jnp.concatenate(parts, axis=1)
        y = jnp.dot(h, w_ref[...], preferred_element_type=jnp.float32) * dinv
        for c in range(nc_out):
            out_ref[c] = y[:, c * F:(c + 1) * F]

    def call(s_pad, xws, dinv, b, a, W):
        return pl.pallas_call(
            mid_body,
            grid=(G,),
            in_specs=[
                pl.BlockSpec((4, R, F), lambda i: (0, i, 0)),
                pl.BlockSpec((4, R, F), lambda i: (0, i, 0)),
                pl.BlockSpec((R, 1), lambda i: (i, 0)),
                pl.BlockSpec((512,), lambda i: (0,)),
                pl.BlockSpec((1, 1), lambda i: (0, 0)),
                pl.BlockSpec((512, nc_out * F), lambda i: (0, 0)),
            ],
            out_specs=pl.BlockSpec((nc_out, R, F), lambda i: (0, i, 0)),
            out_shape=jax.ShapeDtypeStruct((nc_out, N, F), jnp.float32),
            compiler_params=pltpu.CompilerParams(
                dimension_semantics=("arbitrary",)),
        )(s_pad, xws, dinv, b, a, W)

    return call


_tc_mid4 = _make_mid(4)
_tc_mid2 = _make_mid(2)


def _fin_body(s_ref, xw_ref, dinv_ref, b_ref, out_ref):
    dinv = dinv_ref[...]
    for c in range(2):
        out_ref[:, c * F:(c + 1) * F] = (
            (s_ref[c] + xw_ref[c]) * dinv + b_ref[pl.ds(c * F, F)])


def _tc_final(s_pad, xws, dinv, b):
    return pl.pallas_call(
        _fin_body,
        grid=(G,),
        in_specs=[
            pl.BlockSpec((2, R, F), lambda i: (0, i, 0)),
            pl.BlockSpec((2, R, F), lambda i: (0, i, 0)),
            pl.BlockSpec((R, 1), lambda i: (i, 0)),
            pl.BlockSpec((256,), lambda i: (0,)),
        ],
        out_specs=pl.BlockSpec((R, 256), lambda i: (i, 0)),
        out_shape=jax.ShapeDtypeStruct((N, 256), jnp.float32),
        compiler_params=pltpu.CompilerParams(
            dimension_semantics=("arbitrary",)),
    )(s_pad, xws, dinv, b)


def kernel(X, edge_index, W_mlp0, b_mlp0, a_mlp0, W_mlp1, b_mlp1,
           Wg0, bg0, Wg1, bg1, Wg2, bg2, a_gcn):
    ei = edge_index.astype(jnp.int32)
    row = jnp.concatenate([ei[0], jnp.zeros((EP - E,), jnp.int32)])
    col = jnp.concatenate([ei[1], jnp.full((EP - E,), NACC - 1, jnp.int32)])
    offs = (jnp.arange(4, dtype=jnp.int32) * N)[:, None]
    rowc = (row[None, :] + offs).reshape(4, 16, NBG, BG)
    col16 = col.reshape(16, NBG, BG)
    col32 = col.reshape(32, NBD, B)
    zeros_bf = jnp.zeros((B, F), jnp.float32)
    zeros_g = jnp.zeros((BG, F), jnp.float32)
    ones_b = jnp.ones((B, F), jnp.float32)
    a0 = jnp.reshape(a_mlp0, (1, 1))
    ag = jnp.reshape(a_gcn, (1, 1))

    degp = _deg(col32, ones_b, zeros_bf)
    xws0, dinv = _tc_mlp(X, W_mlp0, b_mlp0, a0, W_mlp1, b_mlp1, Wg0, degp)
    s0 = _prop4(xws0.reshape(4 * N, F), rowc, col16, zeros_g)
    xws1 = _tc_mid4(s0, xws0, dinv, bg0, ag, Wg1)
    s1 = _prop4(xws1.reshape(4 * N, F), rowc, col16, zeros_g)
    xws2 = _tc_mid2(s1, xws1, dinv, bg1, ag, Wg2)
    s2 = _prop2(xws2.reshape(2 * N, F), rowc, col16, zeros_g)
    return _tc_final(s2, xws2, dinv, bg2)


# speedup vs baseline: 1.0184x; 1.0184x over previous
import functools

import jax
import jax.numpy as jnp
from jax import lax
from jax.experimental import pallas as pl
from jax.experimental.pallas import tpu as pltpu
from jax.experimental.pallas import tpu_sc as plsc

N = 10000
E = 160000
B = 128
NB = 80
NBD = 40
EP = 163840
NACC = 10240
STRIPE = NACC // 16
F = 128
R = 1000
G = N // R

_mesh = plsc.VectorSubcoreMesh(
    core_axis_name="c", subcore_axis_name="s", num_cores=2, num_subcores=16)


BG = 128
NBUF = 2
NBG = EP // (16 * BG)
QN = 40
NQ = NBG // QN


def _make_prop(nchunk):
    chunks_of_core = {
        0: tuple(c for c in range(nchunk) if c % 2 == 0),
        1: tuple(c for c in range(nchunk) if c % 2 == 1),
    }

    @functools.partial(
        pl.kernel,
        out_type=jax.ShapeDtypeStruct((nchunk, NACC, F), jnp.float32),
        mesh=_mesh,
        scratch_types=[
            pltpu.VMEM_SHARED((NACC, F), jnp.float32),
            pltpu.VMEM((QN, BG), jnp.int32),
            pltpu.VMEM((QN, BG), jnp.int32),
            [pltpu.VMEM((BG, F), jnp.float32) for _ in range(NBUF)],
            [pltpu.SemaphoreType.DMA for _ in range(NBUF)],
            [pltpu.SemaphoreType.DMA for _ in range(NBUF)],
        ],
    )
    def prop(xws_hbm, rowc_hbm, col_hbm, zeros_hbm, out_hbm,
             acc_sh, row_v, col_v, bufs, gsems, ssems):
        cid = lax.axis_index("c")
        sid = lax.axis_index("s")
        base = sid * STRIPE
        for core in (0, 1):
            for c in chunks_of_core[core]:
                @pl.when(cid == core)
                def _(c=c):
                    pltpu.sync_copy(zeros_hbm, bufs[0])
                    for z in range(STRIPE // BG):
                        pltpu.sync_copy(
                            bufs[0], acc_sh.at[pl.ds(base + z * BG, BG)])
                    plsc.subcore_barrier()
                    for q in range(NQ):
                        pltpu.sync_copy(
                            rowc_hbm.at[c, sid, pl.ds(q * QN, QN)], row_v)
                        pltpu.sync_copy(
                            col_hbm.at[sid, pl.ds(q * QN, QN)], col_v)

                        def group(i, carry):
                            j = NBUF * i
                            gds = [
                                pltpu.async_copy(
                                    xws_hbm.at[row_v.at[j + k]],
                                    bufs[k], gsems[k])
                                for k in range(NBUF)]
                            sds = []
                            for k in range(NBUF):
                                gds[k].wait()
                                sds.append(pltpu.async_copy(
                                    bufs[k], acc_sh.at[col_v.at[j + k]],
                                    ssems[k], add=True))
                            for k in range(NBUF):
                                sds[k].wait()
                            return carry

                        lax.fori_loop(0, QN // NBUF, group, 0)
                    plsc.subcore_barrier()
                    for z in range(STRIPE // BG):
                        pltpu.sync_copy(
                            acc_sh.at[pl.ds(base + z * BG, BG)], bufs[0])
                        pltpu.sync_copy(
                            bufs[0], out_hbm.at[c, pl.ds(base + z * BG, BG)])

    return prop


_prop4 = _make_prop(4)
_prop2 = _make_prop(2)


@functools.partial(
    pl.kernel,
    out_type=jax.ShapeDtypeStruct((2, NACC, F), jnp.float32),
    mesh=_mesh,
    scratch_types=[
        pltpu.VMEM_SHARED((NACC, F), jnp.float32),
        pltpu.VMEM((NBD, B), jnp.int32),
        pltpu.VMEM((B, F), jnp.float32),
        pltpu.VMEM((B, F), jnp.float32),
    ],
)
def _deg(col32_hbm, ones_hbm, zeros_hbm, out_hbm, acc_sh, col_v, ones_v, wb_v):
    cid = lax.axis_index("c")
    sid = lax.axis_index("s")
    wid = sid * 2 + cid
    base = sid * STRIPE
    pltpu.sync_copy(col32_hbm.at[wid], col_v)
    pltpu.sync_copy(zeros_hbm, wb_v)
    for z in range(STRIPE // B):
        pltpu.sync_copy(wb_v, acc_sh.at[pl.ds(base + z * B, B)])
    pltpu.sync_copy(ones_hbm, ones_v)
    plsc.subcore_barrier()

    def body(j, carry):
        pltpu.sync_copy(ones_v, acc_sh.at[col_v.at[j]], add=True)
        return carry

    lax.fori_loop(0, NBD, body, 0)
    plsc.subcore_barrier()
    for z in range(STRIPE // B):
        pltpu.sync_copy(acc_sh.at[pl.ds(base + z * B, B)], wb_v)
        pltpu.sync_copy(wb_v, out_hbm.at[cid, pl.ds(base + z * B, B)])


def _mlp_body(x_ref, w0_ref, b0_ref, a0_ref, w1_ref, b1_ref, wg0_ref, deg_ref,
              out_ref, dinv_ref):
    x = x_ref[...]
    a = a0_ref[0, 0]
    h = jnp.dot(x, w0_ref[...], preferred_element_type=jnp.float32) + b0_ref[...]
    h = jnp.where(h >= 0, h, a * h)
    h = jnp.dot(h, w1_ref[...], preferred_element_type=jnp.float32) + b1_ref[...]
    y = jnp.dot(h, wg0_ref[...], preferred_element_type=jnp.float32)
    dinv = lax.rsqrt(deg_ref[0, :, 0:1] + deg_ref[1, :, 0:1] + 1.0)
    dinv_ref[...] = dinv
    y = y * dinv
    for c in range(4):
        out_ref[c] = y[:, c * F:(c + 1) * F]


def _tc_mlp(X, W0, b0, a0, W1, b1, Wg0, degp):
    return pl.pallas_call(
        _mlp_body,
        grid=(G,),
        in_specs=[
            pl.BlockSpec((R, 256), lambda i: (i, 0)),
            pl.BlockSpec((256, 512), lambda i: (0, 0)),
            pl.BlockSpec((512,), lambda i: (0,)),
            pl.BlockSpec((1, 1), lambda i: (0, 0)),
            pl.BlockSpec((512, 512), lambda i: (0, 0)),
            pl.BlockSpec((512,), lambda i: (0,)),
            pl.BlockSpec((512, 512), lambda i: (0, 0)),
            pl.BlockSpec((2, R, F), lambda i: (0, i, 0)),
        ],
        out_specs=[
            pl.BlockSpec((4, R, F), lambda i: (0, i, 0)),
            pl.BlockSpec((R, 1), lambda i: (i, 0)),
        ],
        out_shape=[
            jax.ShapeDtypeStruct((4, N, F), jnp.float32),
            jax.ShapeDtypeStruct((N, 1), jnp.float32),
        ],
        compiler_params=pltpu.CompilerParams(
            dimension_semantics=("arbitrary",)),
    )(X, W0, b0, a0, W1, b1, Wg0, degp)


def _make_mid(nc_out):
    def mid_body(s_ref, xw_ref, dinv_ref, b_ref, a_ref, w_ref, out_ref):
        dinv = dinv_ref[...]
        a = a_ref[0, 0]
        parts = []
        for c in range(4):
            t = (s_ref[c] + xw_ref[c]) * dinv + b_ref[pl.ds(c * F, F)]
            parts.append(jnp.where(t >= 0, t, a * t))
        h = jnp.concatenate(parts, axis=1)
        y = jnp.dot(h, w_ref[...], preferred_element_type=jnp.float32) * dinv
        for c in range(nc_out):
            out_ref[c] = y[:, c * F:(c + 1) * F]

    def call(s_pad, xws, dinv, b, a, W):
        return pl.pallas_call(
            mid_body,
            grid=(G,),
            in_specs=[
                pl.BlockSpec((4, R, F), lambda i: (0, i, 0)),
                pl.BlockSpec((4, R, F), lambda i: (0, i, 0)),
                pl.BlockSpec((R, 1), lambda i: (i, 0)),
                pl.BlockSpec((512,), lambda i: (0,)),
                pl.BlockSpec((1, 1), lambda i: (0, 0)),
                pl.BlockSpec((512, nc_out * F), lambda i: (0, 0)),
            ],
            out_specs=pl.BlockSpec((nc_out, R, F), lambda i: (0, i, 0)),
            out_shape=jax.ShapeDtypeStruct((nc_out, N, F), jnp.float32),
            compiler_params=pltpu.CompilerParams(
                dimension_semantics=("arbitrary",)),
        )(s_pad, xws, dinv, b, a, W)

    return call


_tc_mid4 = _make_mid(4)
_tc_mid2 = _make_mid(2)


def _fin_body(s_ref, xw_ref, dinv_ref, b_ref, out_ref):
    dinv = dinv_ref[...]
    for c in range(2):
        out_ref[:, c * F:(c + 1) * F] = (
            (s_ref[c] + xw_ref[c]) * dinv + b_ref[pl.ds(c * F, F)])


def _tc_final(s_pad, xws, dinv, b):
    return pl.pallas_call(
        _fin_body,
        grid=(G,),
        in_specs=[
            pl.BlockSpec((2, R, F), lambda i: (0, i, 0)),
            pl.BlockSpec((2, R, F), lambda i: (0, i, 0)),
            pl.BlockSpec((R, 1), lambda i: (i, 0)),
            pl.BlockSpec((256,), lambda i: (0,)),
        ],
        out_specs=pl.BlockSpec((R, 256), lambda i: (i, 0)),
        out_shape=jax.ShapeDtypeStruct((N, 256), jnp.float32),
        compiler_params=pltpu.CompilerParams(
            dimension_semantics=("arbitrary",)),
    )(s_pad, xws, dinv, b)


def kernel(X, edge_index, W_mlp0, b_mlp0, a_mlp0, W_mlp1, b_mlp1,
           Wg0, bg0, Wg1, bg1, Wg2, bg2, a_gcn):
    ei = edge_index.astype(jnp.int32)
    row = jnp.concatenate([ei[0], jnp.zeros((EP - E,), jnp.int32)])
    col = jnp.concatenate([ei[1], jnp.full((EP - E,), NACC - 1, jnp.int32)])
    offs = (jnp.arange(4, dtype=jnp.int32) * N)[:, None]
    rowc = (row[None, :] + offs).reshape(4, 16, NBG, BG)
    col16 = col.reshape(16, NBG, BG)
    col32 = col.reshape(32, NBD, B)
    zeros_bf = jnp.zeros((B, F), jnp.float32)
    zeros_g = jnp.zeros((BG, F), jnp.float32)
    ones_b = jnp.ones((B, F), jnp.float32)
    a0 = jnp.reshape(a_mlp0, (1, 1))
    ag = jnp.reshape(a_gcn, (1, 1))

    degp = _deg(col32, ones_b, zeros_bf)
    xws0, dinv = _tc_mlp(X, W_mlp0, b_mlp0, a0, W_mlp1, b_mlp1, Wg0, degp)
    s0 = _prop4(xws0.reshape(4 * N, F), rowc, col16, zeros_g)
    xws1 = _tc_mid4(s0, xws0, dinv, bg0, ag, Wg1)
    s1 = _prop4(xws1.reshape(4 * N, F), rowc, col16, zeros_g)
    xws2 = _tc_mid2(s1, xws1, dinv, bg1, ag, Wg2)
    s2 = _prop2(xws2.reshape(2 * N, F), rowc, col16, zeros_g)
    return _tc_final(s2, xws2, dinv, bg2)


# speedup vs baseline: 1.0839x; 1.0642x over previous
import functools

import jax
import jax.numpy as jnp
from jax import lax
from jax.experimental import pallas as pl
from jax.experimental.pallas import tpu as pltpu
from jax.experimental.pallas import tpu_sc as plsc

N = 10000
E = 160000
B = 128
NB = 80
NBD = 40
EP = 163840
NACC = 10240
STRIPE = NACC // 16
F = 128
R = 1000
G = N // R

_mesh = plsc.VectorSubcoreMesh(
    core_axis_name="c", subcore_axis_name="s", num_cores=2, num_subcores=16)


BG = 128
NBUF = 2
NBG = EP // (16 * BG)
QN = 40
NQ = NBG // QN


def _make_prop(nchunk):
    chunks_of_core = {
        0: tuple(c for c in range(nchunk) if c % 2 == 0),
        1: tuple(c for c in range(nchunk) if c % 2 == 1),
    }

    @functools.partial(
        pl.kernel,
        out_type=jax.ShapeDtypeStruct((nchunk, NACC, F), jnp.float32),
        mesh=_mesh,
        scratch_types=[
            pltpu.VMEM_SHARED((NACC, F), jnp.float32),
            pltpu.VMEM((QN, BG), jnp.int32),
            pltpu.VMEM((QN, BG), jnp.int32),
            [pltpu.VMEM((BG, F), jnp.float32) for _ in range(NBUF)],
            [pltpu.SemaphoreType.DMA for _ in range(NBUF)],
            [pltpu.SemaphoreType.DMA for _ in range(NBUF)],
        ],
    )
    def prop(xws_hbm, rowc_hbm, col_hbm, zeros_hbm, out_hbm,
             acc_sh, row_v, col_v, bufs, gsems, ssems):
        cid = lax.axis_index("c")
        sid = lax.axis_index("s")
        base = sid * STRIPE
        for core in (0, 1):
            for c in chunks_of_core[core]:
                @pl.when(cid == core)
                def _(c=c):
                    pltpu.sync_copy(zeros_hbm, acc_sh.at[pl.ds(base, STRIPE)])
                    plsc.subcore_barrier()
                    for q in range(NQ):
                        pltpu.sync_copy(
                            rowc_hbm.at[c, sid, pl.ds(q * QN, QN)], row_v)
                        pltpu.sync_copy(
                            col_hbm.at[sid, pl.ds(q * QN, QN)], col_v)

                        def group(i, carry):
                            j = NBUF * i
                            gds = [
                                pltpu.async_copy(
                                    xws_hbm.at[row_v.at[j + k]],
                                    bufs[k], gsems[k])
                                for k in range(NBUF)]
                            sds = []
                            for k in range(NBUF):
                                gds[k].wait()
                                sds.append(pltpu.async_copy(
                                    bufs[k], acc_sh.at[col_v.at[j + k]],
                                    ssems[k], add=True))
                            for k in range(NBUF):
                                sds[k].wait()
                            return carry

                        lax.fori_loop(0, QN // NBUF, group, 0)
                    plsc.subcore_barrier()
                    pltpu.sync_copy(acc_sh.at[pl.ds(base, STRIPE)],
                                    out_hbm.at[c, pl.ds(base, STRIPE)])

    return prop


_prop4 = _make_prop(4)
_prop2 = _make_prop(2)


@functools.partial(
    pl.kernel,
    out_type=jax.ShapeDtypeStruct((2, NACC, F), jnp.float32),
    mesh=_mesh,
    scratch_types=[
        pltpu.VMEM_SHARED((NACC, F), jnp.float32),
        pltpu.VMEM((NBD, B), jnp.int32),
        pltpu.VMEM((B, F), jnp.float32),
    ],
)
def _deg(col32_hbm, ones_hbm, zeros_hbm, out_hbm, acc_sh, col_v, ones_v):
    cid = lax.axis_index("c")
    sid = lax.axis_index("s")
    wid = sid * 2 + cid
    base = sid * STRIPE
    pltpu.sync_copy(col32_hbm.at[wid], col_v)
    pltpu.sync_copy(zeros_hbm, acc_sh.at[pl.ds(base, STRIPE)])
    pltpu.sync_copy(ones_hbm, ones_v)
    plsc.subcore_barrier()

    def body(j, carry):
        pltpu.sync_copy(ones_v, acc_sh.at[col_v.at[j]], add=True)
        return carry

    lax.fori_loop(0, NBD, body, 0)
    plsc.subcore_barrier()
    pltpu.sync_copy(acc_sh.at[pl.ds(base, STRIPE)],
                    out_hbm.at[cid, pl.ds(base, STRIPE)])


def _mlp_body(x_ref, w0_ref, b0_ref, a0_ref, w1_ref, b1_ref, wg0_ref, deg_ref,
              out_ref, dinv_ref):
    x = x_ref[...]
    a = a0_ref[0, 0]
    h = jnp.dot(x, w0_ref[...], preferred_element_type=jnp.float32) + b0_ref[...]
    h = jnp.where(h >= 0, h, a * h)
    h = jnp.dot(h, w1_ref[...], preferred_element_type=jnp.float32) + b1_ref[...]
    y = jnp.dot(h, wg0_ref[...], preferred_element_type=jnp.float32)
    dinv = lax.rsqrt(deg_ref[0, :, 0:1] + deg_ref[1, :, 0:1] + 1.0)
    dinv_ref[...] = dinv
    y = y * dinv
    for c in range(4):
        out_ref[c] = y[:, c * F:(c + 1) * F]


def _tc_mlp(X, W0, b0, a0, W1, b1, Wg0, degp):
    return pl.pallas_call(
        _mlp_body,
        grid=(G,),
        in_specs=[
            pl.BlockSpec((R, 256), lambda i: (i, 0)),
            pl.BlockSpec((256, 512), lambda i: (0, 0)),
            pl.BlockSpec((512,), lambda i: (0,)),
            pl.BlockSpec((1, 1), lambda i: (0, 0)),
            pl.BlockSpec((512, 512), lambda i: (0, 0)),
            pl.BlockSpec((512,), lambda i: (0,)),
            pl.BlockSpec((512, 512), lambda i: (0, 0)),
            pl.BlockSpec((2, R, F), lambda i: (0, i, 0)),
        ],
        out_specs=[
            pl.BlockSpec((4, R, F), lambda i: (0, i, 0)),
            pl.BlockSpec((R, 1), lambda i: (i, 0)),
        ],
        out_shape=[
            jax.ShapeDtypeStruct((4, N, F), jnp.float32),
            jax.ShapeDtypeStruct((N, 1), jnp.float32),
        ],
        compiler_params=pltpu.CompilerParams(
            dimension_semantics=("arbitrary",)),
    )(X, W0, b0, a0, W1, b1, Wg0, degp)


def _make_mid(nc_out):
    def mid_body(s_ref, xw_ref, dinv_ref, b_ref, a_ref, w_ref, out_ref):
        dinv = dinv_ref[...]
        a = a_ref[0, 0]
        parts = []
        for c in range(4):
            t = (s_ref[c] + xw_ref[c]) * dinv + b_ref[pl.ds(c * F, F)]
            parts.append(jnp.where(t >= 0, t, a * t))
        h = jnp.concatenate(parts, axis=1)
        y = jnp.dot(h, w_ref[...], preferred_element_type=jnp.float32) * dinv
        for c in range(nc_out):
            out_ref[c] = y[:, c * F:(c + 1) * F]

    def call(s_pad, xws, dinv, b, a, W):
        return pl.pallas_call(
            mid_body,
            grid=(G,),
            in_specs=[
                pl.BlockSpec((4, R, F), lambda i: (0, i, 0)),
                pl.BlockSpec((4, R, F), lambda i: (0, i, 0)),
                pl.BlockSpec((R, 1), lambda i: (i, 0)),
                pl.BlockSpec((512,), lambda i: (0,)),
                pl.BlockSpec((1, 1), lambda i: (0, 0)),
                pl.BlockSpec((512, nc_out * F), lambda i: (0, 0)),
            ],
            out_specs=pl.BlockSpec((nc_out, R, F), lambda i: (0, i, 0)),
            out_shape=jax.ShapeDtypeStruct((nc_out, N, F), jnp.float32),
            compiler_params=pltpu.CompilerParams(
                dimension_semantics=("arbitrary",)),
        )(s_pad, xws, dinv, b, a, W)

    return call


_tc_mid4 = _make_mid(4)
_tc_mid2 = _make_mid(2)


def _fin_body(s_ref, xw_ref, dinv_ref, b_ref, out_ref):
    dinv = dinv_ref[...]
    for c in range(2):
        out_ref[:, c * F:(c + 1) * F] = (
            (s_ref[c] + xw_ref[c]) * dinv + b_ref[pl.ds(c * F, F)])


def _tc_final(s_pad, xws, dinv, b):
    return pl.pallas_call(
        _fin_body,
        grid=(G,),
        in_specs=[
            pl.BlockSpec((2, R, F), lambda i: (0, i, 0)),
            pl.BlockSpec((2, R, F), lambda i: (0, i, 0)),
            pl.BlockSpec((R, 1), lambda i: (i, 0)),
            pl.BlockSpec((256,), lambda i: (0,)),
        ],
        out_specs=pl.BlockSpec((R, 256), lambda i: (i, 0)),
        out_shape=jax.ShapeDtypeStruct((N, 256), jnp.float32),
        compiler_params=pltpu.CompilerParams(
            dimension_semantics=("arbitrary",)),
    )(s_pad, xws, dinv, b)


def kernel(X, edge_index, W_mlp0, b_mlp0, a_mlp0, W_mlp1, b_mlp1,
           Wg0, bg0, Wg1, bg1, Wg2, bg2, a_gcn):
    ei = edge_index.astype(jnp.int32)
    row = jnp.concatenate([ei[0], jnp.zeros((EP - E,), jnp.int32)])
    col = jnp.concatenate([ei[1], jnp.full((EP - E,), NACC - 1, jnp.int32)])
    offs = (jnp.arange(4, dtype=jnp.int32) * N)[:, None]
    rowc = (row[None, :] + offs).reshape(4, 16, NBG, BG)
    col16 = col.reshape(16, NBG, BG)
    col32 = col.reshape(32, NBD, B)
    zeros_s = jnp.zeros((STRIPE, F), jnp.float32)
    ones_b = jnp.ones((B, F), jnp.float32)
    a0 = jnp.reshape(a_mlp0, (1, 1))
    ag = jnp.reshape(a_gcn, (1, 1))

    degp = _deg(col32, ones_b, zeros_s)
    xws0, dinv = _tc_mlp(X, W_mlp0, b_mlp0, a0, W_mlp1, b_mlp1, Wg0, degp)
    s0 = _prop4(xws0.reshape(4 * N, F), rowc, col16, zeros_s)
    xws1 = _tc_mid4(s0, xws0, dinv, bg0, ag, Wg1)
    s1 = _prop4(xws1.reshape(4 * N, F), rowc, col16, zeros_s)
    xws2 = _tc_mid2(s1, xws1, dinv, bg1, ag, Wg2)
    s2 = _prop2(xws2.reshape(2 * N, F), rowc, col16, zeros_s)
    return _tc_final(s2, xws2, dinv, bg2)


# speedup vs baseline: 1.0866x; 1.0025x over previous
import functools

import jax
import jax.numpy as jnp
from jax import lax
from jax.experimental import pallas as pl
from jax.experimental.pallas import tpu as pltpu
from jax.experimental.pallas import tpu_sc as plsc

N = 10000
E = 160000
B = 128
NB = 80
NBD = 40
EP = 163840
NACC = 10240
STRIPE = NACC // 16
F = 128
R = 1000
G = N // R

_mesh = plsc.VectorSubcoreMesh(
    core_axis_name="c", subcore_axis_name="s", num_cores=2, num_subcores=16)


BG = 128
NBUF = 2
NBG = EP // (16 * BG)
QN = 40
NQ = NBG // QN


def _make_prop(nchunk):
    chunks_of_core = {
        0: tuple(c for c in range(nchunk) if c % 2 == 0),
        1: tuple(c for c in range(nchunk) if c % 2 == 1),
    }

    @functools.partial(
        pl.kernel,
        out_type=jax.ShapeDtypeStruct((nchunk, NACC, F), jnp.float32),
        mesh=_mesh,
        scratch_types=[
            pltpu.VMEM_SHARED((NACC, F), jnp.float32),
            pltpu.VMEM((QN, BG), jnp.int32),
            pltpu.VMEM((QN, BG), jnp.int32),
            [pltpu.VMEM((BG, F), jnp.float32) for _ in range(NBUF)],
            [pltpu.SemaphoreType.DMA for _ in range(NBUF)],
            [pltpu.SemaphoreType.DMA for _ in range(NBUF)],
        ],
    )
    def prop(xws_hbm, rowc_hbm, col_hbm, zeros_hbm, out_hbm,
             acc_sh, row_v, col_v, bufs, gsems, ssems):
        cid = lax.axis_index("c")
        sid = lax.axis_index("s")
        base = sid * STRIPE
        for core in (0, 1):
            for c in chunks_of_core[core]:
                @pl.when(cid == core)
                def _(c=c):
                    pltpu.sync_copy(zeros_hbm, acc_sh.at[pl.ds(base, STRIPE)])
                    plsc.subcore_barrier()
                    for q in range(NQ):
                        pltpu.sync_copy(
                            rowc_hbm.at[c, sid, pl.ds(q * QN, QN)], row_v)
                        pltpu.sync_copy(
                            col_hbm.at[sid, pl.ds(q * QN, QN)], col_v)

                        def group(i, carry):
                            j = NBUF * i
                            gds = [
                                pltpu.async_copy(
                                    xws_hbm.at[row_v.at[j + k]],
                                    bufs[k], gsems[k])
                                for k in range(NBUF)]
                            sds = []
                            for k in range(NBUF):
                                gds[k].wait()
                                sds.append(pltpu.async_copy(
                                    bufs[k], acc_sh.at[col_v.at[j + k]],
                                    ssems[k], add=True))
                            for k in range(NBUF):
                                sds[k].wait()
                            return carry

                        lax.fori_loop(0, QN // NBUF, group, 0)
                    plsc.subcore_barrier()
                    pltpu.sync_copy(acc_sh.at[pl.ds(base, STRIPE)],
                                    out_hbm.at[c, pl.ds(base, STRIPE)])

    return prop


_prop4 = _make_prop(4)
_prop2 = _make_prop(2)


@functools.partial(
    pl.kernel,
    out_type=jax.ShapeDtypeStruct((2, NACC, F), jnp.float32),
    mesh=_mesh,
    scratch_types=[
        pltpu.VMEM_SHARED((NACC, F), jnp.float32),
        pltpu.VMEM((NBD, B), jnp.int32),
        pltpu.VMEM((B, F), jnp.float32),
    ],
)
def _deg(col32_hbm, ones_hbm, zeros_hbm, out_hbm, acc_sh, col_v, ones_v):
    cid = lax.axis_index("c")
    sid = lax.axis_index("s")
    wid = sid * 2 + cid
    base = sid * STRIPE
    pltpu.sync_copy(col32_hbm.at[wid], col_v)
    pltpu.sync_copy(zeros_hbm, acc_sh.at[pl.ds(base, STRIPE)])
    pltpu.sync_copy(ones_hbm, ones_v)
    plsc.subcore_barrier()

    def body(j, carry):
        pltpu.sync_copy(ones_v, acc_sh.at[col_v.at[j]], add=True)
        return carry

    lax.fori_loop(0, NBD, body, 0)
    plsc.subcore_barrier()
    pltpu.sync_copy(acc_sh.at[pl.ds(base, STRIPE)],
                    out_hbm.at[cid, pl.ds(base, STRIPE)])


def _mlp_body(x_ref, w0_ref, b0_ref, a0_ref, w1_ref, b1_ref, wg0_ref,
              out_ref):
    x = x_ref[...]
    a = a0_ref[0, 0]
    h = jnp.dot(x, w0_ref[...], preferred_element_type=jnp.float32) + b0_ref[...]
    h = jnp.where(h >= 0, h, a * h)
    h = jnp.dot(h, w1_ref[...], preferred_element_type=jnp.float32) + b1_ref[...]
    y = jnp.dot(h, wg0_ref[...], preferred_element_type=jnp.float32)
    for c in range(4):
        out_ref[c] = y[:, c * F:(c + 1) * F]


def _tc_mlp(X, W0, b0, a0, W1, b1, Wg0):
    return pl.pallas_call(
        _mlp_body,
        grid=(G,),
        in_specs=[
            pl.BlockSpec((R, 256), lambda i: (i, 0)),
            pl.BlockSpec((256, 512), lambda i: (0, 0)),
            pl.BlockSpec((512,), lambda i: (0,)),
            pl.BlockSpec((1, 1), lambda i: (0, 0)),
            pl.BlockSpec((512, 512), lambda i: (0, 0)),
            pl.BlockSpec((512,), lambda i: (0,)),
            pl.BlockSpec((512, 512), lambda i: (0, 0)),
        ],
        out_specs=pl.BlockSpec((4, R, F), lambda i: (0, i, 0)),
        out_shape=jax.ShapeDtypeStruct((4, N, F), jnp.float32),
        compiler_params=pltpu.CompilerParams(
            dimension_semantics=("arbitrary",)),
    )(X, W0, b0, a0, W1, b1, Wg0)


def _scale_body(y_ref, deg_ref, out_ref, dinv_ref):
    dinv = lax.rsqrt(deg_ref[0, :, 0:1] + deg_ref[1, :, 0:1] + 1.0)
    dinv_ref[...] = dinv
    for c in range(4):
        out_ref[c] = y_ref[c] * dinv


def _tc_scale(y, degp):
    return pl.pallas_call(
        _scale_body,
        grid=(G,),
        in_specs=[
            pl.BlockSpec((4, R, F), lambda i: (0, i, 0)),
            pl.BlockSpec((2, R, F), lambda i: (0, i, 0)),
        ],
        out_specs=[
            pl.BlockSpec((4, R, F), lambda i: (0, i, 0)),
            pl.BlockSpec((R, 1), lambda i: (i, 0)),
        ],
        out_shape=[
            jax.ShapeDtypeStruct((4, N, F), jnp.float32),
            jax.ShapeDtypeStruct((N, 1), jnp.float32),
        ],
        compiler_params=pltpu.CompilerParams(
            dimension_semantics=("arbitrary",)),
    )(y, degp)


def _make_mid(nc_out):
    def mid_body(s_ref, xw_ref, dinv_ref, b_ref, a_ref, w_ref, out_ref):
        dinv = dinv_ref[...]
        a = a_ref[0, 0]
        parts = []
        for c in range(4):
            t = (s_ref[c] + xw_ref[c]) * dinv + b_ref[pl.ds(c * F, F)]
            parts.append(jnp.where(t >= 0, t, a * t))
        h = jnp.concatenate(parts, axis=1)
        y = jnp.dot(h, w_ref[...], preferred_element_type=jnp.float32) * dinv
        for c in range(nc_out):
            out_ref[c] = y[:, c * F:(c + 1) * F]

    def call(s_pad, xws, dinv, b, a, W):
        return pl.pallas_call(
            mid_body,
            grid=(G,),
            in_specs=[
                pl.BlockSpec((4, R, F), lambda i: (0, i, 0)),
                pl.BlockSpec((4, R, F), lambda i: (0, i, 0)),
                pl.BlockSpec((R, 1), lambda i: (i, 0)),
                pl.BlockSpec((512,), lambda i: (0,)),
                pl.BlockSpec((1, 1), lambda i: (0, 0)),
                pl.BlockSpec((512, nc_out * F), lambda i: (0, 0)),
            ],
            out_specs=pl.BlockSpec((nc_out, R, F), lambda i: (0, i, 0)),
            out_shape=jax.ShapeDtypeStruct((nc_out, N, F), jnp.float32),
            compiler_params=pltpu.CompilerParams(
                dimension_semantics=("arbitrary",)),
        )(s_pad, xws, dinv, b, a, W)

    return call


_tc_mid4 = _make_mid(4)
_tc_mid2 = _make_mid(2)


def _fin_body(s_ref, xw_ref, dinv_ref, b_ref, out_ref):
    dinv = dinv_ref[...]
    for c in range(2):
        out_ref[:, c * F:(c + 1) * F] = (
            (s_ref[c] + xw_ref[c]) * dinv + b_ref[pl.ds(c * F, F)])


def _tc_final(s_pad, xws, dinv, b):
    return pl.pallas_call(
        _fin_body,
        grid=(G,),
        in_specs=[
            pl.BlockSpec((2, R, F), lambda i: (0, i, 0)),
            pl.BlockSpec((2, R, F), lambda i: (0, i, 0)),
            pl.BlockSpec((R, 1), lambda i: (i, 0)),
            pl.BlockSpec((256,), lambda i: (0,)),
        ],
        out_specs=pl.BlockSpec((R, 256), lambda i: (i, 0)),
        out_shape=jax.ShapeDtypeStruct((N, 256), jnp.float32),
        compiler_params=pltpu.CompilerParams(
            dimension_semantics=("arbitrary",)),
    )(s_pad, xws, dinv, b)


def kernel(X, edge_index, W_mlp0, b_mlp0, a_mlp0, W_mlp1, b_mlp1,
           Wg0, bg0, Wg1, bg1, Wg2, bg2, a_gcn):
    ei = edge_index.astype(jnp.int32)
    row = jnp.concatenate([ei[0], jnp.zeros((EP - E,), jnp.int32)])
    col = jnp.concatenate([ei[1], jnp.full((EP - E,), NACC - 1, jnp.int32)])
    offs = (jnp.arange(4, dtype=jnp.int32) * N)[:, None]
    rowc = (row[None, :] + offs).reshape(4, 16, NBG, BG)
    col16 = col.reshape(16, NBG, BG)
    col32 = col.reshape(32, NBD, B)
    zeros_s = jnp.zeros((STRIPE, F), jnp.float32)
    ones_b = jnp.ones((B, F), jnp.float32)
    a0 = jnp.reshape(a_mlp0, (1, 1))
    ag = jnp.reshape(a_gcn, (1, 1))

    degp = _deg(col32, ones_b, zeros_s)
    y0 = _tc_mlp(X, W_mlp0, b_mlp0, a0, W_mlp1, b_mlp1, Wg0)
    xws0, dinv = _tc_scale(y0, degp)
    s0 = _prop4(xws0.reshape(4 * N, F), rowc, col16, zeros_s)
    xws1 = _tc_mid4(s0, xws0, dinv, bg0, ag, Wg1)
    s1 = _prop4(xws1.reshape(4 * N, F), rowc, col16, zeros_s)
    xws2 = _tc_mid2(s1, xws1, dinv, bg1, ag, Wg2)
    s2 = _prop2(xws2.reshape(2 * N, F), rowc, col16, zeros_s)
    return _tc_final(s2, xws2, dinv, bg2)
